# baseline (speedup 1.0000x reference)
import functools

import jax
import jax.numpy as jnp
from jax.experimental import pallas as pl
from jax.experimental.pallas import tpu as pltpu

_B, _N0, _K = 4, 512, 16


def _mm_body(x_ref, w_ref, b_ref, o_ref):
    o_ref[...] = jnp.dot(x_ref[...], w_ref[...],
                         preferred_element_type=jnp.float32) + b_ref[...]


def _pallas_mm(x, W, b):
    n, cin = x.shape
    cout = W.shape[1]
    blk = 512
    grid = (n // blk,)
    return pl.pallas_call(
        _mm_body,
        grid=grid,
        in_specs=[
            pl.BlockSpec((blk, cin), lambda i: (i, 0)),
            pl.BlockSpec((cin, cout), lambda i: (0, 0)),
            pl.BlockSpec((cout,), lambda i: (0,)),
        ],
        out_specs=pl.BlockSpec((blk, cout), lambda i: (i, 0)),
        out_shape=jax.ShapeDtypeStruct((n, cout), jnp.float32),
    )(x, W, b)


def _knn_body(n, R, prow_ref, pcolT_ref, out_ref):
    b = pl.program_id(0)
    i = pl.program_id(1)
    pr = prow_ref[0]
    pcT = pcolT_ref[0]
    sqr = jnp.sum(pr * pr, axis=1, keepdims=True)
    sqc = jnp.sum(pcT * pcT, axis=0, keepdims=True)
    dot = jax.lax.dot_general(pr, pcT, (((1,), (0,)), ((), ())),
                              preferred_element_type=jnp.float32)
    d2 = sqr + sqc - 2.0 * dot
    rowid = i * R + jax.lax.broadcasted_iota(jnp.int32, (R, 1), 0)
    colid = jax.lax.broadcasted_iota(jnp.int32, (R, n), 1)
    d2 = d2 + jnp.where(colid == rowid, jnp.float32(1e10), jnp.float32(0.0))
    cols = []
    for t in range(_K):
        m = jnp.min(d2, axis=1, keepdims=True)
        j = jnp.min(jnp.where(d2 == m, colid, jnp.int32(n)), axis=1, keepdims=True)
        cols.append(j)
        d2 = jnp.where(colid == j, jnp.float32(3.4028235e38), d2)
    out_ref[0] = jnp.concatenate(cols, axis=1) + b * n


def _knn_graph(points, b, n, k):
    R = 8
    pts3 = points.reshape(b, n, 3)
    ptsT = pts3.transpose(0, 2, 1)
    nbr = pl.pallas_call(
        functools.partial(_knn_body, n, R),
        grid=(b, n // R),
        in_specs=[
            pl.BlockSpec((1, R, 3), lambda bb, i: (bb, i, 0)),
            pl.BlockSpec((1, 3, n), lambda bb, i: (bb, 0, 0)),
        ],
        out_specs=pl.BlockSpec((1, R, k), lambda bb, i: (bb, i, 0)),
        out_shape=jax.ShapeDtypeStruct((b, n, k), jnp.int32),
    )(pts3, ptsT)
    offs = (jnp.arange(b, dtype=jnp.int32) * n)[:, None, None]
    src = nbr.reshape(-1)
    dst = (jnp.broadcast_to(jnp.arange(n, dtype=jnp.int32)[None, :, None], (b, n, k)) + offs).reshape(-1)
    return src, dst


def _transformer_conv(x, src, dst, Wq, bq, Wk, bk, Wv, bv, Ws, bs):
    n = x.shape[0]
    d = Wq.shape[1]
    q = _pallas_mm(x, Wq, bq)
    kk = _pallas_mm(x, Wk, bk)
    v = _pallas_mm(x, Wv, bv)
    logits = jnp.sum(q[dst] * kk[src], axis=-1) / jnp.sqrt(jnp.float32(d))
    amax = jax.lax.stop_gradient(jax.ops.segment_max(logits, dst, num_segments=n))
    ex = jnp.exp(logits - amax[dst])
    den = jax.ops.segment_sum(ex, dst, num_segments=n)
    alpha = ex / (den[dst] + 1e-16)
    out = jax.ops.segment_sum(v[src] * alpha[:, None], dst, num_segments=n)
    return out + _pallas_mm(x, Ws, bs)


def _layer(p, name, x, src, dst):
    return _transformer_conv(x, src, dst, p[name + '_Wq'], p[name + '_bq'], p[name + '_Wk'], p[name + '_bk'], p[name + '_Wv'], p[name + '_bv'], p[name + '_Ws'], p[name + '_bs'])


def kernel(features, points, b1t1_Wq, b1t1_bq, b1t1_Wk, b1t1_bk, b1t1_Wv, b1t1_bv, b1t1_Ws, b1t1_bs, b1t2_Wq, b1t2_bq, b1t2_Wk, b1t2_bk, b1t2_Wv, b1t2_bv, b1t2_Ws, b1t2_bs, b1t3_Wq, b1t3_bq, b1t3_Wk, b1t3_bk, b1t3_Wv, b1t3_bv, b1t3_Ws, b1t3_bs, b1p_Wq, b1p_bq, b1p_Wk, b1p_bk, b1p_Wv, b1p_bv, b1p_Ws, b1p_bs, b2t1_Wq, b2t1_bq, b2t1_Wk, b2t1_bk, b2t1_Wv, b2t1_bv, b2t1_Ws, b2t1_bs, b2t2_Wq, b2t2_bq, b2t2_Wk, b2t2_bk, b2t2_Wv, b2t2_bv, b2t2_Ws, b2t2_bs, b2t3_Wq, b2t3_bq, b2t3_Wk, b2t3_bk, b2t3_Wv, b2t3_bv, b2t3_Ws, b2t3_bs, b2p_Wq, b2p_bq, b2p_Wk, b2p_bk, b2p_Wv, b2p_bv, b2p_Ws, b2p_bs, b3t1_Wq, b3t1_bq, b3t1_Wk, b3t1_bk, b3t1_Wv, b3t1_bv, b3t1_Ws, b3t1_bs, b3t2_Wq, b3t2_bq, b3t2_Wk, b3t2_bk, b3t2_Wv, b3t2_bv, b3t2_Ws, b3t2_bs, b3t3_Wq, b3t3_bq, b3t3_Wk, b3t3_bk, b3t3_Wv, b3t3_bv, b3t3_Ws, b3t3_bs):
    p = dict(locals())
    features = p['features'].reshape(-1, 64)
    points = p['points'].reshape(-1, 3)
    src, dst = _knn_graph(points, _B, _N0, _K)
    b1 = _layer(p, 'b1t1', features, src, dst)
    b1 = _layer(p, 'b1t2', b1, src, dst)
    b1 = _layer(p, 'b1t3', b1, src, dst)
    pts = _layer(p, 'b1p', jnp.concatenate([points, b1], axis=-1), src, dst)
    b1 = jnp.repeat(b1.reshape(_B, _N0, 64), 3, axis=1).reshape(-1, 64)
    pts = jnp.repeat(pts.reshape(_B, _N0, 3), 3, axis=1).reshape(-1, 3)
    n2 = _N0 * 3
    src, dst = _knn_graph(pts, _B, n2, _K)
    b2 = _layer(p, 'b2t1', b1, src, dst)
    b2 = _layer(p, 'b2t2', b2, src, dst)
    b2 = _layer(p, 'b2t3', b2, src, dst)
    pts = _layer(p, 'b2p', jnp.concatenate([pts, b2], axis=-1), src, dst)
    b2 = jnp.repeat(b2.reshape(_B, n2, 32), 3, axis=1).reshape(-1, 32)
    pts = jnp.repeat(pts.reshape(_B, n2, 3), 3, axis=1).reshape(-1, 3)
    n3 = n2 * 3
    src, dst = _knn_graph(pts, _B, n3, _K)
    b3 = _layer(p, 'b3t1', b2, src, dst)
    b3 = _layer(p, 'b3t2', b3, src, dst)
    b3 = _layer(p, 'b3t3', b3, src, dst)
    return b3.reshape(_B, -1, 3)
